# baseline (speedup 1.0000x reference)
import functools

import jax
import jax.numpy as jnp
from jax import lax
from jax.experimental import pallas as pl
from jax.experimental.pallas import tpu as pltpu
from jax.experimental.pallas import tpu_sc as plsc

NC = 2
NS = 16
NW = NC * NS
CHUNK = 80
ZROWS = 32


def _sc_aggregate(x, edge_flat, e, with_deg):
    n, d = x.shape
    dt = x.dtype
    ept = e // NW
    nchunk = ept // CHUNK
    ra = NS * (16 if dt == jnp.bfloat16 else 8)
    n_pad = ((n + ra - 1) // ra) * ra
    rpt = n_pad // NS

    mesh = plsc.VectorSubcoreMesh(core_axis_name="c", subcore_axis_name="s")

    assert nchunk % 2 == 1 and nchunk >= 3

    NB = 4

    out_type = [jax.ShapeDtypeStruct((NC, n_pad, d), dt)]
    scratch = []
    scratch += [pltpu.VMEM((CHUNK,), jnp.int32) for _ in range(NB)]
    scratch += [pltpu.VMEM((CHUNK,), jnp.int32) for _ in range(NB)]
    scratch += [pltpu.VMEM((CHUNK, d), dt) for _ in range(NB)]
    scratch += [
        pltpu.VMEM((ZROWS, d), dt),
        pltpu.VMEM_SHARED((n_pad, d), dt),
    ]
    scratch += [pltpu.SemaphoreType.DMA for _ in range(4 * NB)]
    if with_deg:
        out_type.append(jax.ShapeDtypeStruct((NC * n_pad,), jnp.float32))
        scratch += [
            pltpu.VMEM((CHUNK,), jnp.float32),
            pltpu.VMEM((d,), jnp.float32),
            pltpu.VMEM((rpt,), jnp.float32),
            pltpu.VMEM_SHARED((n_pad,), jnp.float32),
        ]

    def body(*refs):
        nin, nout = 2, len(out_type)
        x_hbm, edge_hbm = refs[:nin]
        agg_out = refs[nin]
        deg_out = refs[nin + 1] if with_deg else None
        sc = list(refs[nin + nout:])
        srcb = tuple(sc.pop(0) for _ in range(NB))
        dstb = tuple(sc.pop(0) for _ in range(NB))
        rowsb = tuple(sc.pop(0) for _ in range(NB))
        zrow_v = sc.pop(0)
        agg_s = sc.pop(0)
        isem = tuple(sc.pop(0) for _ in range(NB))
        dsem = tuple(sc.pop(0) for _ in range(NB))
        gsem = tuple(sc.pop(0) for _ in range(NB))
        ssem = tuple(sc.pop(0) for _ in range(NB))
        if with_deg:
            ones_v, zf32_v, dtmp_v, deg_s = sc

        cid = lax.axis_index("c")
        sid = lax.axis_index("s")
        wid = cid * NS + sid
        r0 = sid * rpt
        e0 = wid * ept

        lanes = 32 if dt == jnp.bfloat16 else 16
        for i in range(ZROWS):
            for j in range(d // lanes):
                zrow_v[i, pl.ds(j * lanes, lanes)] = jnp.zeros((lanes,), dt)

        nz, zrem = rpt // ZROWS, rpt % ZROWS

        def zcopy_body(k, carry):
            pltpu.async_copy(zrow_v, agg_s.at[pl.ds(r0 + k * ZROWS, ZROWS)],
                             ssem[0])
            return carry
        lax.fori_loop(0, nz, zcopy_body, 0)
        if zrem:
            pltpu.async_copy(zrow_v.at[pl.ds(0, zrem)],
                             agg_s.at[pl.ds(r0 + nz * ZROWS, zrem)], ssem[0])

        if with_deg:
            def ones_body(i, carry):
                ones_v[pl.ds(i * 16, 16)] = jnp.ones((16,), jnp.float32)
                return carry
            lax.fori_loop(0, CHUNK // 16, ones_body, 0)

            def zf32_body(i, carry):
                zf32_v[pl.ds(i * 16, 16)] = jnp.zeros((16,), jnp.float32)
                return carry
            lax.fori_loop(0, d // 16, zf32_body, 0)

            def zdeg_body(k, carry):
                pltpu.async_copy(zf32_v, deg_s.at[pl.ds(r0 + k * d, d)],
                                 ssem[1])
                return carry
            lax.fori_loop(0, rpt // d, zdeg_body, 0)
            drem = rpt % d
            if drem:
                pltpu.async_copy(zf32_v.at[pl.ds(0, drem)],
                                 deg_s.at[pl.ds(r0 + (rpt // d) * d, drem)],
                                 ssem[1])

        def zdrain_body(k, carry):
            pltpu.make_async_copy(zrow_v,
                                  agg_s.at[pl.ds(r0 + k * ZROWS, ZROWS)],
                                  ssem[0]).wait()
            return carry
        lax.fori_loop(0, nz, zdrain_body, 0)
        if zrem:
            pltpu.make_async_copy(zrow_v.at[pl.ds(0, zrem)],
                                  agg_s.at[pl.ds(r0 + nz * ZROWS, zrem)],
                                  ssem[0]).wait()
        if with_deg:
            def zdeg_drain(k, carry):
                pltpu.make_async_copy(zf32_v,
                                      deg_s.at[pl.ds(r0 + k * d, d)],
                                      ssem[1]).wait()
                return carry
            lax.fori_loop(0, rpt // d, zdeg_drain, 0)
            if drem:
                pltpu.make_async_copy(
                    zf32_v.at[pl.ds(0, drem)],
                    deg_s.at[pl.ds(r0 + (rpt // d) * d, drem)],
                    ssem[1]).wait()

        plsc.subcore_barrier()

        def issue_src(g, b):
            pltpu.async_copy(edge_hbm.at[pl.ds(e0 + g * CHUNK, CHUNK)],
                             srcb[b], isem[b])

        def wait_src(g, b):
            pltpu.make_async_copy(edge_hbm.at[pl.ds(e0 + g * CHUNK, CHUNK)],
                                  srcb[b], isem[b]).wait()

        def issue_dst(g, b):
            pltpu.async_copy(edge_hbm.at[pl.ds(e + e0 + g * CHUNK, CHUNK)],
                             dstb[b], dsem[b])

        def wait_dst(g, b):
            pltpu.make_async_copy(
                edge_hbm.at[pl.ds(e + e0 + g * CHUNK, CHUNK)],
                dstb[b], dsem[b]).wait()

        def issue_rows(b):
            pltpu.async_copy(x_hbm.at[srcb[b]], rowsb[b], gsem[b])

        def wait_rows(b):
            pltpu.make_async_copy(x_hbm.at[srcb[b]], rowsb[b],
                                  gsem[b]).wait()

        def start_scatter(b):
            pltpu.async_copy(rowsb[b], agg_s.at[dstb[b]], ssem[b], add=True)
            if with_deg:
                pltpu.async_copy(ones_v, deg_s.at[dstb[b]], ssem[b], add=True)

        def wait_scatter(b):
            pltpu.make_async_copy(rowsb[b], agg_s.at[dstb[b]],
                                  ssem[b]).wait()
            if with_deg:
                pltpu.make_async_copy(ones_v, deg_s.at[dstb[b]],
                                      ssem[b]).wait()

        issue_src(0, 0)
        issue_src(1, 1)
        issue_src(2, 2)
        issue_dst(0, 0)
        issue_dst(1, 1)
        wait_src(0, 0)
        issue_rows(0)
        wait_src(1, 1)
        issue_rows(1)

        def do_visit(v, j, guards=(True, True, True)):
            g_issue, s_issue, d_issue = guards
            if g_issue:
                wait_src(v + 2, (j + 2) % NB)
                issue_rows((j + 2) % NB)
            if s_issue:
                issue_src(v + 3, (j + 3) % NB)
            if d_issue:
                issue_dst(v + 2, (j + 2) % NB)
            wait_rows(j)
            wait_dst(v, j)
            start_scatter(j)

        do_visit(0, 0)
        do_visit(1, 1)

        nloop = (nchunk - 7) // NB

        def main_body(i, carry):
            v0 = 2 + NB * i
            for j in range(NB):
                wait_scatter(j % NB)
                do_visit(v0 + j, (2 + j) % NB)
            return carry
        lax.fori_loop(0, nloop, main_body, 0)

        for v in range(2 + NB * nloop, nchunk):
            b = v % NB
            wait_scatter((v - 2) % NB)
            do_visit(v, b, guards=(v + 2 < nchunk, v + 3 < nchunk,
                                   v + 2 < nchunk))
        wait_scatter((nchunk - 2) % NB)
        wait_scatter((nchunk - 1) % NB)

        plsc.subcore_barrier()

        pltpu.sync_copy(agg_s.at[pl.ds(r0, rpt)],
                        agg_out.at[cid, pl.ds(r0, rpt)])
        if with_deg:
            pltpu.sync_copy(deg_s.at[pl.ds(r0, rpt)], dtmp_v)
            pltpu.sync_copy(dtmp_v, deg_out.at[pl.ds(cid * n_pad + r0, rpt)])

    fn = pl.kernel(body, mesh=mesh, out_type=out_type, scratch_types=scratch)
    outs = fn(x, edge_flat)
    if with_deg:
        return outs[0], outs[1]
    return outs[0], None


def _tc_matmul(x, w, b):
    def body(x_ref, w_ref, b_ref, o_ref):
        o_ref[...] = jnp.dot(x_ref[...], w_ref[...],
                             preferred_element_type=jnp.float32) + b_ref[...]
    return pl.pallas_call(
        body,
        out_shape=jax.ShapeDtypeStruct((x.shape[0], w.shape[1]), jnp.float32),
    )(x, w, b)


def _tc_layer1(selfp, parts, degT, wn, g, be):
    def body(s_ref, p_ref, d_ref, wn_ref, g_ref, be_ref, o_ref):
        n = s_ref.shape[0]
        p = p_ref[...].astype(jnp.float32)
        deg = jnp.sum(d_ref[...], axis=1, keepdims=True)[:n]
        mean = (p[0, :n] + p[1, :n]) / jnp.maximum(deg, 1.0)
        y = s_ref[...] + jnp.dot(mean.astype(jnp.bfloat16),
                                 wn_ref[...].astype(jnp.bfloat16),
                                 preferred_element_type=jnp.float32)
        mu = jnp.mean(y, axis=0, keepdims=True)
        var = jnp.mean(y * y, axis=0, keepdims=True) - mu * mu
        h = g_ref[...] * (y - mu) / jnp.sqrt(var + 1e-5) + be_ref[...]
        o_ref[...] = jnp.maximum(h, 0.0)

    return pl.pallas_call(
        body, out_shape=jax.ShapeDtypeStruct(selfp.shape, jnp.float32),
    )(selfp, parts, degT, wn, g, be)


def _tc_layer2_head(selfp, parts, degT, wn, g, be,
                    wc1, bc1, gc1, bec1, wc2, bc2, gc2, bec2, wc3p, bc3p):
    n = selfp.shape[0]

    def bn(y, gg, bb):
        mu = jnp.mean(y, axis=0, keepdims=True)
        var = jnp.mean(y * y, axis=0, keepdims=True) - mu * mu
        return gg * (y - mu) / jnp.sqrt(var + 1e-5) + bb

    def body(s_ref, p_ref, d_ref, wn_ref, g_ref, be_ref,
             wc1_ref, bc1_ref, gc1_ref, bec1_ref,
             wc2_ref, bc2_ref, gc2_ref, bec2_ref,
             wc3_ref, bc3_ref, o_ref):
        nn = s_ref.shape[0]
        p = p_ref[...].astype(jnp.float32)
        deg = jnp.sum(d_ref[...], axis=1, keepdims=True)[:nn]
        mean = (p[0, :nn] + p[1, :nn]) / jnp.maximum(deg, 1.0)

        def bdot(a, w):
            return jnp.dot(a.astype(jnp.bfloat16), w.astype(jnp.bfloat16),
                           preferred_element_type=jnp.float32)

        y = s_ref[...] + bdot(mean, wn_ref[...])
        h2 = bn(y, g_ref[...], be_ref[...])
        c1 = jnp.maximum(bn(bdot(h2, wc1_ref[...]) + bc1_ref[...],
                            gc1_ref[...], bec1_ref[...]), 0.0)
        c2 = jnp.maximum(bn(bdot(c1, wc2_ref[...]) + bc2_ref[...],
                            gc2_ref[...], bec2_ref[...]), 0.0)
        o_ref[...] = (jnp.dot(c2, wc3_ref[...],
                              preferred_element_type=jnp.float32)
                      + bc3_ref[...])

    return pl.pallas_call(
        body, out_shape=jax.ShapeDtypeStruct((n, 1), jnp.float32),
    )(selfp, parts, degT, wn, g, be,
      wc1, bc1, gc1, bec1, wc2, bc2, gc2, bec2, wc3p, bc3p)


def kernel(node_features, edge_index, W_self1, W_neigh1, b1, g1, be1,
           W_self2, W_neigh2, b2, g2, be2, Wc1, bc1, gc1, bec1,
           Wc2, bc2, gc2, bec2, Wc3, bc3):
    x = node_features
    e = edge_index.shape[1]
    edge_flat = edge_index.reshape(-1)
    selfp1 = _tc_matmul(x, W_self1, b1.reshape(1, -1))
    parts1, deg_parts = _sc_aggregate(x, edge_flat, e, with_deg=True)
    degT = jnp.transpose(deg_parts.reshape(NC, -1))
    h1 = _tc_layer1(selfp1, parts1, degT, W_neigh1,
                    g1.reshape(1, -1), be1.reshape(1, -1))
    selfp2 = _tc_matmul(h1, W_self2, b2.reshape(1, -1))
    parts2, _ = _sc_aggregate(h1, edge_flat, e, with_deg=False)
    return _tc_layer2_head(selfp2, parts2, degT, W_neigh2,
                           g2.reshape(1, -1), be2.reshape(1, -1),
                           Wc1, bc1.reshape(1, -1), gc1.reshape(1, -1),
                           bec1.reshape(1, -1),
                           Wc2, bc2.reshape(1, -1), gc2.reshape(1, -1),
                           bec2.reshape(1, -1), Wc3, bc3.reshape(1, -1))


# speedup vs baseline: 1.0255x; 1.0255x over previous
import functools

import jax
import jax.numpy as jnp
from jax import lax
from jax.experimental import pallas as pl
from jax.experimental.pallas import tpu as pltpu
from jax.experimental.pallas import tpu_sc as plsc

NC = 2
NS = 16
NW = NC * NS
CHUNK = 80
ZROWS = 32


def _sc_aggregate(x, edge_flat, e, with_deg):
    n, d = x.shape
    dt = x.dtype
    ept = e // NW
    nchunk = ept // CHUNK
    ra = NS * (16 if dt == jnp.bfloat16 else 8)
    n_pad = ((n + ra - 1) // ra) * ra
    rpt = n_pad // NS

    mesh = plsc.VectorSubcoreMesh(core_axis_name="c", subcore_axis_name="s")

    assert nchunk % 2 == 1 and nchunk >= 3

    NB = 4

    out_type = [jax.ShapeDtypeStruct((NC, n_pad, d), dt)]
    scratch = []
    scratch += [pltpu.VMEM((CHUNK,), jnp.int32) for _ in range(NB)]
    scratch += [pltpu.VMEM((CHUNK,), jnp.int32) for _ in range(NB)]
    scratch += [pltpu.VMEM((CHUNK, d), dt) for _ in range(NB)]
    scratch += [
        pltpu.VMEM((ZROWS, d), dt),
        pltpu.VMEM_SHARED((n_pad, d), dt),
    ]
    scratch += [pltpu.SemaphoreType.DMA for _ in range(4 * NB)]
    if with_deg:
        out_type.append(jax.ShapeDtypeStruct((NC * n_pad,), jnp.float32))
        scratch += [
            pltpu.VMEM((CHUNK,), jnp.float32),
            pltpu.VMEM((d,), jnp.float32),
            pltpu.VMEM((rpt,), jnp.float32),
            pltpu.VMEM_SHARED((n_pad,), jnp.float32),
        ]

    def body(*refs):
        nin, nout = 2, len(out_type)
        x_hbm, edge_hbm = refs[:nin]
        agg_out = refs[nin]
        deg_out = refs[nin + 1] if with_deg else None
        sc = list(refs[nin + nout:])
        srcb = tuple(sc.pop(0) for _ in range(NB))
        dstb = tuple(sc.pop(0) for _ in range(NB))
        rowsb = tuple(sc.pop(0) for _ in range(NB))
        zrow_v = sc.pop(0)
        agg_s = sc.pop(0)
        isem = tuple(sc.pop(0) for _ in range(NB))
        dsem = tuple(sc.pop(0) for _ in range(NB))
        gsem = tuple(sc.pop(0) for _ in range(NB))
        ssem = tuple(sc.pop(0) for _ in range(NB))
        if with_deg:
            ones_v, zf32_v, dtmp_v, deg_s = sc

        cid = lax.axis_index("c")
        sid = lax.axis_index("s")
        wid = cid * NS + sid
        r0 = sid * rpt
        e0 = wid * ept

        lanes = 32 if dt == jnp.bfloat16 else 16
        for i in range(ZROWS):
            for j in range(d // lanes):
                zrow_v[i, pl.ds(j * lanes, lanes)] = jnp.zeros((lanes,), dt)

        nz, zrem = rpt // ZROWS, rpt % ZROWS

        def zcopy_body(k, carry):
            pltpu.async_copy(zrow_v, agg_s.at[pl.ds(r0 + k * ZROWS, ZROWS)],
                             ssem[0])
            return carry
        lax.fori_loop(0, nz, zcopy_body, 0)
        if zrem:
            pltpu.async_copy(zrow_v.at[pl.ds(0, zrem)],
                             agg_s.at[pl.ds(r0 + nz * ZROWS, zrem)], ssem[0])

        if with_deg:
            def ones_body(i, carry):
                ones_v[pl.ds(i * 16, 16)] = jnp.ones((16,), jnp.float32)
                return carry
            lax.fori_loop(0, CHUNK // 16, ones_body, 0)

            def zf32_body(i, carry):
                zf32_v[pl.ds(i * 16, 16)] = jnp.zeros((16,), jnp.float32)
                return carry
            lax.fori_loop(0, d // 16, zf32_body, 0)

            def zdeg_body(k, carry):
                pltpu.async_copy(zf32_v, deg_s.at[pl.ds(r0 + k * d, d)],
                                 ssem[1])
                return carry
            lax.fori_loop(0, rpt // d, zdeg_body, 0)
            drem = rpt % d
            if drem:
                pltpu.async_copy(zf32_v.at[pl.ds(0, drem)],
                                 deg_s.at[pl.ds(r0 + (rpt // d) * d, drem)],
                                 ssem[1])

        def zdrain_body(k, carry):
            pltpu.make_async_copy(zrow_v,
                                  agg_s.at[pl.ds(r0 + k * ZROWS, ZROWS)],
                                  ssem[0]).wait()
            return carry
        lax.fori_loop(0, nz, zdrain_body, 0)
        if zrem:
            pltpu.make_async_copy(zrow_v.at[pl.ds(0, zrem)],
                                  agg_s.at[pl.ds(r0 + nz * ZROWS, zrem)],
                                  ssem[0]).wait()
        if with_deg:
            def zdeg_drain(k, carry):
                pltpu.make_async_copy(zf32_v,
                                      deg_s.at[pl.ds(r0 + k * d, d)],
                                      ssem[1]).wait()
                return carry
            lax.fori_loop(0, rpt // d, zdeg_drain, 0)
            if drem:
                pltpu.make_async_copy(
                    zf32_v.at[pl.ds(0, drem)],
                    deg_s.at[pl.ds(r0 + (rpt // d) * d, drem)],
                    ssem[1]).wait()

        plsc.subcore_barrier()

        def issue_src(g, b):
            pltpu.async_copy(edge_hbm.at[pl.ds(e0 + g * CHUNK, CHUNK)],
                             srcb[b], isem[b])

        def wait_src(g, b):
            pltpu.make_async_copy(edge_hbm.at[pl.ds(e0 + g * CHUNK, CHUNK)],
                                  srcb[b], isem[b]).wait()

        def issue_dst(g, b):
            pltpu.async_copy(edge_hbm.at[pl.ds(e + e0 + g * CHUNK, CHUNK)],
                             dstb[b], dsem[b])

        def wait_dst(g, b):
            pltpu.make_async_copy(
                edge_hbm.at[pl.ds(e + e0 + g * CHUNK, CHUNK)],
                dstb[b], dsem[b]).wait()

        def issue_rows(b):
            pltpu.async_copy(x_hbm.at[srcb[b]], rowsb[b], gsem[b])

        def wait_rows(b):
            pltpu.make_async_copy(x_hbm.at[srcb[b]], rowsb[b],
                                  gsem[b]).wait()

        def start_scatter(b):
            pltpu.async_copy(rowsb[b], agg_s.at[dstb[b]], ssem[b], add=True)
            if with_deg:
                pltpu.async_copy(ones_v, deg_s.at[dstb[b]], ssem[b], add=True)

        def wait_scatter(b):
            pltpu.make_async_copy(rowsb[b], agg_s.at[dstb[b]],
                                  ssem[b]).wait()
            if with_deg:
                pltpu.make_async_copy(ones_v, deg_s.at[dstb[b]],
                                      ssem[b]).wait()

        issue_src(0, 0)
        issue_src(1, 1)
        issue_src(2, 2)
        issue_dst(0, 0)
        issue_dst(1, 1)
        wait_src(0, 0)
        issue_rows(0)
        wait_src(1, 1)
        issue_rows(1)

        def do_visit(v, j, guards=(True, True, True)):
            g_issue, s_issue, d_issue = guards
            if g_issue:
                wait_src(v + 2, (j + 2) % NB)
                issue_rows((j + 2) % NB)
            if s_issue:
                issue_src(v + 3, (j + 3) % NB)
            if d_issue:
                issue_dst(v + 2, (j + 2) % NB)
            wait_rows(j)
            wait_dst(v, j)
            start_scatter(j)

        do_visit(0, 0)
        do_visit(1, 1)

        nloop = (nchunk - 7) // NB

        def main_body(i, carry):
            v0 = 2 + NB * i
            for j in range(NB):
                wait_scatter(j % NB)
                do_visit(v0 + j, (2 + j) % NB)
            return carry
        lax.fori_loop(0, nloop, main_body, 0)

        for v in range(2 + NB * nloop, nchunk):
            b = v % NB
            wait_scatter((v - 2) % NB)
            do_visit(v, b, guards=(v + 2 < nchunk, v + 3 < nchunk,
                                   v + 2 < nchunk))
        wait_scatter((nchunk - 2) % NB)
        wait_scatter((nchunk - 1) % NB)

        plsc.subcore_barrier()

        pltpu.sync_copy(agg_s.at[pl.ds(r0, rpt)],
                        agg_out.at[cid, pl.ds(r0, rpt)])
        if with_deg:
            pltpu.sync_copy(deg_s.at[pl.ds(r0, rpt)], dtmp_v)
            pltpu.sync_copy(dtmp_v, deg_out.at[pl.ds(cid * n_pad + r0, rpt)])

    fn = pl.kernel(body, mesh=mesh, out_type=out_type, scratch_types=scratch)
    outs = fn(x, edge_flat)
    if with_deg:
        return outs[0], outs[1]
    return outs[0], None


def _tc_matmul(x, w, b):
    def body(x_ref, w_ref, b_ref, o_ref):
        o_ref[...] = jnp.dot(x_ref[...], w_ref[...],
                             preferred_element_type=jnp.float32) + b_ref[...]
    return pl.pallas_call(
        body,
        out_shape=jax.ShapeDtypeStruct((x.shape[0], w.shape[1]), jnp.float32),
    )(x, w, b)


def _tc_layer1(selfp, parts, degT, wn, g, be):
    def body(s_ref, p_ref, d_ref, wn_ref, g_ref, be_ref, o_ref):
        n = s_ref.shape[0]
        npad = d_ref.shape[0] // NC
        p = p_ref[...].astype(jnp.float32)
        dv = d_ref[...]
        deg = jnp.transpose((dv[0:n] + dv[npad:npad + n]).reshape(1, n))
        mean = (p[0, :n] + p[1, :n]) / jnp.maximum(deg, 1.0)
        y = s_ref[...] + jnp.dot(mean, wn_ref[...],
                                 preferred_element_type=jnp.float32)
        mu = jnp.mean(y, axis=0, keepdims=True)
        var = jnp.mean(y * y, axis=0, keepdims=True) - mu * mu
        h = g_ref[...] * (y - mu) / jnp.sqrt(var + 1e-5) + be_ref[...]
        o_ref[...] = jnp.maximum(h, 0.0)

    return pl.pallas_call(
        body, out_shape=jax.ShapeDtypeStruct(selfp.shape, jnp.float32),
    )(selfp, parts, degT, wn, g, be)


def _tc_layer2_head(selfp, parts, degT, wn, g, be,
                    wc1, bc1, gc1, bec1, wc2, bc2, gc2, bec2, wc3p, bc3p):
    n = selfp.shape[0]

    def bn(y, gg, bb):
        mu = jnp.mean(y, axis=0, keepdims=True)
        var = jnp.mean(y * y, axis=0, keepdims=True) - mu * mu
        return gg * (y - mu) / jnp.sqrt(var + 1e-5) + bb

    def body(s_ref, p_ref, d_ref, wn_ref, g_ref, be_ref,
             wc1_ref, bc1_ref, gc1_ref, bec1_ref,
             wc2_ref, bc2_ref, gc2_ref, bec2_ref,
             wc3_ref, bc3_ref, o_ref):
        nn = s_ref.shape[0]
        npad = d_ref.shape[0] // NC
        p = p_ref[...].astype(jnp.float32)
        dv = d_ref[...]
        deg = jnp.transpose((dv[0:nn] + dv[npad:npad + nn]).reshape(1, nn))
        mean = (p[0, :nn] + p[1, :nn]) / jnp.maximum(deg, 1.0)
        y = s_ref[...] + jnp.dot(mean, wn_ref[...],
                                 preferred_element_type=jnp.float32)
        h2 = bn(y, g_ref[...], be_ref[...])
        c1 = jnp.maximum(bn(jnp.dot(h2, wc1_ref[...],
                                    preferred_element_type=jnp.float32)
                            + bc1_ref[...], gc1_ref[...], bec1_ref[...]), 0.0)
        c2 = jnp.maximum(bn(jnp.dot(c1, wc2_ref[...],
                                    preferred_element_type=jnp.float32)
                            + bc2_ref[...], gc2_ref[...], bec2_ref[...]), 0.0)
        o_ref[...] = (jnp.dot(c2, wc3_ref[...],
                              preferred_element_type=jnp.float32)
                      + bc3_ref[...])

    return pl.pallas_call(
        body, out_shape=jax.ShapeDtypeStruct((n, 1), jnp.float32),
    )(selfp, parts, degT, wn, g, be,
      wc1, bc1, gc1, bec1, wc2, bc2, gc2, bec2, wc3p, bc3p)


def kernel(node_features, edge_index, W_self1, W_neigh1, b1, g1, be1,
           W_self2, W_neigh2, b2, g2, be2, Wc1, bc1, gc1, bec1,
           Wc2, bc2, gc2, bec2, Wc3, bc3):
    x = node_features
    e = edge_index.shape[1]
    edge_flat = edge_index.reshape(-1)
    selfp1 = _tc_matmul(x, W_self1, b1.reshape(1, -1))
    parts1, deg_parts = _sc_aggregate(x, edge_flat, e, with_deg=True)
    degT = deg_parts
    h1 = _tc_layer1(selfp1, parts1, degT, W_neigh1,
                    g1.reshape(1, -1), be1.reshape(1, -1))
    selfp2 = _tc_matmul(h1, W_self2, b2.reshape(1, -1))
    parts2, _ = _sc_aggregate(h1, edge_flat, e, with_deg=False)
    return _tc_layer2_head(selfp2, parts2, degT, W_neigh2,
                           g2.reshape(1, -1), be2.reshape(1, -1),
                           Wc1, bc1.reshape(1, -1), gc1.reshape(1, -1),
                           bec1.reshape(1, -1),
                           Wc2, bc2.reshape(1, -1), gc2.reshape(1, -1),
                           bec2.reshape(1, -1), Wc3, bc3.reshape(1, -1))


# speedup vs baseline: 1.0265x; 1.0009x over previous
import jax
import jax.numpy as jnp
from jax import lax
from jax.experimental import pallas as pl
from jax.experimental.pallas import tpu as pltpu
from jax.experimental.pallas import tpu_sc as plsc

NC = 2
NS = 16
NW = NC * NS
CHUNK = 80
ZROWS = 32


def _sc_aggregate(x, edge_flat, e, with_deg):
    n, d = x.shape
    dt = x.dtype
    ept = e // NW
    nchunk = ept // CHUNK
    ra = NS * (16 if dt == jnp.bfloat16 else 8)
    n_pad = ((n + ra - 1) // ra) * ra
    rpt = n_pad // NS

    mesh = plsc.VectorSubcoreMesh(core_axis_name="c", subcore_axis_name="s")

    assert nchunk % 2 == 1 and nchunk >= 3

    NB = 4

    out_type = [jax.ShapeDtypeStruct((NC, n_pad, d), dt)]
    scratch = []
    scratch += [pltpu.VMEM((CHUNK,), jnp.int32) for _ in range(NB)]
    scratch += [pltpu.VMEM((CHUNK,), jnp.int32) for _ in range(NB)]
    scratch += [pltpu.VMEM((CHUNK, d), dt) for _ in range(NB)]
    scratch += [
        pltpu.VMEM((ZROWS, d), dt),
        pltpu.VMEM_SHARED((n_pad, d), dt),
    ]
    scratch += [pltpu.SemaphoreType.DMA for _ in range(4 * NB)]
    if with_deg:
        out_type.append(jax.ShapeDtypeStruct((NC * n_pad,), jnp.float32))
        scratch += [
            pltpu.VMEM((CHUNK,), jnp.float32),
            pltpu.VMEM((d,), jnp.float32),
            pltpu.VMEM((rpt,), jnp.float32),
            pltpu.VMEM_SHARED((n_pad,), jnp.float32),
        ]

    def body(*refs):
        nin, nout = 2, len(out_type)
        x_hbm, edge_hbm = refs[:nin]
        agg_out = refs[nin]
        deg_out = refs[nin + 1] if with_deg else None
        sc = list(refs[nin + nout:])
        srcb = tuple(sc.pop(0) for _ in range(NB))
        dstb = tuple(sc.pop(0) for _ in range(NB))
        rowsb = tuple(sc.pop(0) for _ in range(NB))
        zrow_v = sc.pop(0)
        agg_s = sc.pop(0)
        isem = tuple(sc.pop(0) for _ in range(NB))
        dsem = tuple(sc.pop(0) for _ in range(NB))
        gsem = tuple(sc.pop(0) for _ in range(NB))
        ssem = tuple(sc.pop(0) for _ in range(NB))
        if with_deg:
            ones_v, zf32_v, dtmp_v, deg_s = sc

        cid = lax.axis_index("c")
        sid = lax.axis_index("s")
        wid = cid * NS + sid
        r0 = sid * rpt
        e0 = wid * ept

        lanes = 32 if dt == jnp.bfloat16 else 16
        for i in range(ZROWS):
            for j in range(d // lanes):
                zrow_v[i, pl.ds(j * lanes, lanes)] = jnp.zeros((lanes,), dt)

        nz, zrem = rpt // ZROWS, rpt % ZROWS

        def zcopy_body(k, carry):
            pltpu.async_copy(zrow_v, agg_s.at[pl.ds(r0 + k * ZROWS, ZROWS)],
                             ssem[0])
            return carry
        lax.fori_loop(0, nz, zcopy_body, 0)
        if zrem:
            pltpu.async_copy(zrow_v.at[pl.ds(0, zrem)],
                             agg_s.at[pl.ds(r0 + nz * ZROWS, zrem)], ssem[0])

        if with_deg:
            def ones_body(i, carry):
                ones_v[pl.ds(i * 16, 16)] = jnp.ones((16,), jnp.float32)
                return carry
            lax.fori_loop(0, CHUNK // 16, ones_body, 0)

            def zf32_body(i, carry):
                zf32_v[pl.ds(i * 16, 16)] = jnp.zeros((16,), jnp.float32)
                return carry
            lax.fori_loop(0, d // 16, zf32_body, 0)

            def zdeg_body(k, carry):
                pltpu.async_copy(zf32_v, deg_s.at[pl.ds(r0 + k * d, d)],
                                 ssem[1])
                return carry
            lax.fori_loop(0, rpt // d, zdeg_body, 0)
            drem = rpt % d
            if drem:
                pltpu.async_copy(zf32_v.at[pl.ds(0, drem)],
                                 deg_s.at[pl.ds(r0 + (rpt // d) * d, drem)],
                                 ssem[1])

        def zdrain_body(k, carry):
            pltpu.make_async_copy(zrow_v,
                                  agg_s.at[pl.ds(r0 + k * ZROWS, ZROWS)],
                                  ssem[0]).wait()
            return carry
        lax.fori_loop(0, nz, zdrain_body, 0)
        if zrem:
            pltpu.make_async_copy(zrow_v.at[pl.ds(0, zrem)],
                                  agg_s.at[pl.ds(r0 + nz * ZROWS, zrem)],
                                  ssem[0]).wait()
        if with_deg:
            def zdeg_drain(k, carry):
                pltpu.make_async_copy(zf32_v,
                                      deg_s.at[pl.ds(r0 + k * d, d)],
                                      ssem[1]).wait()
                return carry
            lax.fori_loop(0, rpt // d, zdeg_drain, 0)
            if drem:
                pltpu.make_async_copy(
                    zf32_v.at[pl.ds(0, drem)],
                    deg_s.at[pl.ds(r0 + (rpt // d) * d, drem)],
                    ssem[1]).wait()

        plsc.subcore_barrier()

        def issue_src(g, b):
            pltpu.async_copy(edge_hbm.at[pl.ds(e0 + g * CHUNK, CHUNK)],
                             srcb[b], isem[b])

        def wait_src(g, b):
            pltpu.make_async_copy(edge_hbm.at[pl.ds(e0 + g * CHUNK, CHUNK)],
                                  srcb[b], isem[b]).wait()

        def issue_dst(g, b):
            pltpu.async_copy(edge_hbm.at[pl.ds(e + e0 + g * CHUNK, CHUNK)],
                             dstb[b], dsem[b])

        def wait_dst(g, b):
            pltpu.make_async_copy(
                edge_hbm.at[pl.ds(e + e0 + g * CHUNK, CHUNK)],
                dstb[b], dsem[b]).wait()

        def issue_rows(b):
            pltpu.async_copy(x_hbm.at[srcb[b]], rowsb[b], gsem[b])

        def wait_rows(b):
            pltpu.make_async_copy(x_hbm.at[srcb[b]], rowsb[b],
                                  gsem[b]).wait()

        def start_scatter(b):
            pltpu.async_copy(rowsb[b], agg_s.at[dstb[b]], ssem[b], add=True)
            if with_deg:
                pltpu.async_copy(ones_v, deg_s.at[dstb[b]], ssem[b], add=True)

        def wait_scatter(b):
            pltpu.make_async_copy(rowsb[b], agg_s.at[dstb[b]],
                                  ssem[b]).wait()
            if with_deg:
                pltpu.make_async_copy(ones_v, deg_s.at[dstb[b]],
                                      ssem[b]).wait()

        issue_src(0, 0)
        issue_src(1, 1)
        issue_src(2, 2)
        issue_dst(0, 0)
        issue_dst(1, 1)
        wait_src(0, 0)
        issue_rows(0)
        wait_src(1, 1)
        issue_rows(1)

        def do_visit(v, j, guards=(True, True, True)):
            g_issue, s_issue, d_issue = guards
            if g_issue:
                wait_src(v + 2, (j + 2) % NB)
                issue_rows((j + 2) % NB)
            if s_issue:
                issue_src(v + 3, (j + 3) % NB)
            if d_issue:
                issue_dst(v + 2, (j + 2) % NB)
            wait_rows(j)
            wait_dst(v, j)
            start_scatter(j)

        do_visit(0, 0)
        do_visit(1, 1)

        nloop = (nchunk - 7) // NB

        def main_body(i, carry):
            v0 = 2 + NB * i
            for j in range(NB):
                wait_scatter(j % NB)
                do_visit(v0 + j, (2 + j) % NB)
            return carry
        lax.fori_loop(0, nloop, main_body, 0)

        for v in range(2 + NB * nloop, nchunk):
            b = v % NB
            wait_scatter((v - 2) % NB)
            do_visit(v, b, guards=(v + 2 < nchunk, v + 3 < nchunk,
                                   v + 2 < nchunk))
        wait_scatter((nchunk - 2) % NB)
        wait_scatter((nchunk - 1) % NB)

        plsc.subcore_barrier()

        pltpu.sync_copy(agg_s.at[pl.ds(r0, rpt)],
                        agg_out.at[cid, pl.ds(r0, rpt)])
        if with_deg:
            pltpu.sync_copy(deg_s.at[pl.ds(r0, rpt)], dtmp_v)
            pltpu.sync_copy(dtmp_v, deg_out.at[pl.ds(cid * n_pad + r0, rpt)])

    fn = pl.kernel(body, mesh=mesh, out_type=out_type, scratch_types=scratch)
    outs = fn(x, edge_flat)
    if with_deg:
        return outs[0], outs[1]
    return outs[0], None


def _tc_matmul(x, w, b):
    def body(x_ref, w_ref, b_ref, o_ref):
        o_ref[...] = jnp.dot(x_ref[...], w_ref[...],
                             preferred_element_type=jnp.float32) + b_ref[...]
    return pl.pallas_call(
        body,
        out_shape=jax.ShapeDtypeStruct((x.shape[0], w.shape[1]), jnp.float32),
    )(x, w, b)


def _tc_layer1(selfp, parts, degT, wn, g, be):
    def body(s_ref, p_ref, d_ref, wn_ref, g_ref, be_ref, o_ref):
        n = s_ref.shape[0]
        npad = d_ref.shape[0] // NC
        p = p_ref[...].astype(jnp.float32)
        dv = d_ref[...]
        deg = jnp.transpose((dv[0:n] + dv[npad:npad + n]).reshape(1, n))
        mean = (p[0, :n] + p[1, :n]) / jnp.maximum(deg, 1.0)
        y = s_ref[...] + jnp.dot(mean, wn_ref[...],
                                 preferred_element_type=jnp.float32)
        mu = jnp.mean(y, axis=0, keepdims=True)
        var = jnp.mean(y * y, axis=0, keepdims=True) - mu * mu
        h = g_ref[...] * (y - mu) / jnp.sqrt(var + 1e-5) + be_ref[...]
        o_ref[...] = jnp.maximum(h, 0.0)

    return pl.pallas_call(
        body, out_shape=jax.ShapeDtypeStruct(selfp.shape, jnp.float32),
    )(selfp, parts, degT, wn, g, be)


def _tc_layer2_head(selfp, parts, degT, wn, g, be,
                    wc1, bc1, gc1, bec1, wc2, bc2, gc2, bec2, wc3p, bc3p):
    n = selfp.shape[0]

    def bn(y, gg, bb):
        mu = jnp.mean(y, axis=0, keepdims=True)
        var = jnp.mean(y * y, axis=0, keepdims=True) - mu * mu
        return gg * (y - mu) / jnp.sqrt(var + 1e-5) + bb

    def body(s_ref, p_ref, d_ref, wn_ref, g_ref, be_ref,
             wc1_ref, bc1_ref, gc1_ref, bec1_ref,
             wc2_ref, bc2_ref, gc2_ref, bec2_ref,
             wc3_ref, bc3_ref, o_ref):
        nn = s_ref.shape[0]
        npad = d_ref.shape[0] // NC
        p = p_ref[...].astype(jnp.float32)
        dv = d_ref[...]
        deg = jnp.transpose((dv[0:nn] + dv[npad:npad + nn]).reshape(1, nn))
        mean = (p[0, :nn] + p[1, :nn]) / jnp.maximum(deg, 1.0)
        y = s_ref[...] + jnp.dot(mean, wn_ref[...],
                                 preferred_element_type=jnp.float32)
        h2 = bn(y, g_ref[...], be_ref[...])
        c1 = jnp.maximum(bn(jnp.dot(h2, wc1_ref[...],
                                    preferred_element_type=jnp.float32)
                            + bc1_ref[...], gc1_ref[...], bec1_ref[...]), 0.0)
        c2 = jnp.maximum(bn(jnp.dot(c1, wc2_ref[...],
                                    preferred_element_type=jnp.float32)
                            + bc2_ref[...], gc2_ref[...], bec2_ref[...]), 0.0)
        o_ref[...] = (jnp.dot(c2, wc3_ref[...],
                              preferred_element_type=jnp.float32)
                      + bc3_ref[...])

    return pl.pallas_call(
        body, out_shape=jax.ShapeDtypeStruct((n, 1), jnp.float32),
    )(selfp, parts, degT, wn, g, be,
      wc1, bc1, gc1, bec1, wc2, bc2, gc2, bec2, wc3p, bc3p)


def kernel(node_features, edge_index, W_self1, W_neigh1, b1, g1, be1,
           W_self2, W_neigh2, b2, g2, be2, Wc1, bc1, gc1, bec1,
           Wc2, bc2, gc2, bec2, Wc3, bc3):
    x = node_features
    e = edge_index.shape[1]
    edge_flat = edge_index.reshape(-1)
    selfp1 = _tc_matmul(x, W_self1, b1.reshape(1, -1))
    parts1, deg_parts = _sc_aggregate(x, edge_flat, e, with_deg=True)
    degT = deg_parts
    h1 = _tc_layer1(selfp1, parts1, degT, W_neigh1,
                    g1.reshape(1, -1), be1.reshape(1, -1))
    selfp2 = _tc_matmul(h1, W_self2, b2.reshape(1, -1))
    parts2, _ = _sc_aggregate(h1, edge_flat, e, with_deg=False)
    return _tc_layer2_head(selfp2, parts2, degT, W_neigh2,
                           g2.reshape(1, -1), be2.reshape(1, -1),
                           Wc1, bc1.reshape(1, -1), gc1.reshape(1, -1),
                           bec1.reshape(1, -1),
                           Wc2, bc2.reshape(1, -1), gc2.reshape(1, -1),
                           bec2.reshape(1, -1), Wc3, bc3.reshape(1, -1))
